# baseline (speedup 1.0000x reference)
import functools

import jax
import jax.numpy as jnp
from jax import lax
from jax.experimental import pallas as pl
from jax.experimental.pallas import tpu as pltpu
from jax.experimental.pallas import tpu_sc as plsc

NUM_WORKERS = 32
CHUNK = 128
LANES = 16


@functools.partial(jax.jit, static_argnums=(2, 3, 4))
def _sc_embed(idx, table, B, K, N):
    D = table.shape[1]
    chunks_per_batch = N // CHUNK
    batches_per_worker = B // NUM_WORKERS
    total_chunks = batches_per_worker * chunks_per_batch

    mesh = plsc.VectorSubcoreMesh(core_axis_name="c", subcore_axis_name="s")

    @functools.partial(
        pl.kernel,
        out_type=jax.ShapeDtypeStruct((B, N, D), jnp.float32),
        mesh=mesh,
        scratch_types=[
            pltpu.VMEM((K, CHUNK), jnp.int32),
            pltpu.VMEM((CHUNK, D), jnp.float32),
            pltpu.SemaphoreType.DMA,
        ],
        compiler_params=pltpu.CompilerParams(use_tc_tiling_on_sc=False),
    )
    def body(idx_hbm, table_hbm, out_hbm, idx_v, acc_t, sem):
        wid = lax.axis_index("s") * 2 + lax.axis_index("c")

        def chunk_step(i, _):
            b = wid * batches_per_worker + (i // chunks_per_batch)
            base = (i % chunks_per_batch) * CHUNK
            pltpu.sync_copy(idx_hbm.at[b, :, pl.ds(base, CHUNK)], idx_v)
            pltpu.async_copy(table_hbm.at[idx_v.at[0]], acc_t, sem).wait()
            copies = [
                pltpu.async_copy(table_hbm.at[idx_v.at[k]], acc_t, sem,
                                 add=True)
                for k in range(1, K)
            ]
            for cp in copies:
                cp.wait()
            pltpu.sync_copy(acc_t, out_hbm.at[b, pl.ds(base, CHUNK), :])
            return 0

        lax.fori_loop(0, total_chunks, chunk_step, 0, unroll=False)

    return body(idx, table)


def _transpose_block(scale, x_ref, o_ref):
    xb = x_ref[0]
    dd = (xb.shape[1], xb.shape[1])
    rows = jax.lax.broadcasted_iota(jnp.int32, dd, 0)
    cols = jax.lax.broadcasted_iota(jnp.int32, dd, 1)
    ident = jnp.where(rows == cols, scale, 0.0).astype(jnp.float32)
    o_ref[0] = jax.lax.dot_general(
        ident, xb, (((1,), (1,)), ((), ())),
        preferred_element_type=jnp.float32)


@functools.partial(jax.jit, static_argnums=(1, 2, 3, 4))
def _tc_transpose(x, B, N, D, scale):
    NB = 512
    return pl.pallas_call(
        functools.partial(_transpose_block, scale),
        grid=(B, N // NB),
        in_specs=[pl.BlockSpec((1, NB, D), lambda b, n: (b, n, 0))],
        out_specs=pl.BlockSpec((1, D, NB), lambda b, n: (b, 0, n)),
        out_shape=jax.ShapeDtypeStruct((B, D, N), jnp.float32),
    )(x)


def kernel(inputs, table):
    B, K, H, W = inputs.shape
    N = H * W
    D = table.shape[1]
    idx = inputs.reshape(B, K, N).astype(jnp.int32)
    pm = _sc_embed(idx, table, B, K, N)
    out = _tc_transpose(pm, B, N, D, 1.0 / K)
    return out.reshape(B, D, H, W)


# speedup vs baseline: 2.7664x; 2.7664x over previous
import functools

import jax
import jax.numpy as jnp
from jax import lax
from jax.experimental import pallas as pl
from jax.experimental.pallas import tpu as pltpu
from jax.experimental.pallas import tpu_sc as plsc

NUM_WORKERS = 32
L = 16


WP = 65


@functools.partial(jax.jit, static_argnums=(2, 3, 4, 5))
def _sc_embed(idx, table, B, K, H, W):
    D = table.shape[1]
    bpw = B // NUM_WORKERS
    T = bpw * H
    scale = 1.0 / K

    mesh = plsc.VectorSubcoreMesh(core_axis_name="c", subcore_axis_name="s")

    @functools.partial(
        pl.kernel,
        out_type=jax.ShapeDtypeStruct((B, D, H, W), jnp.float32),
        mesh=mesh,
        scratch_types=[
            pltpu.VMEM((2, K, W), jnp.int32),
            pltpu.VMEM((2, W, D), jnp.float32),
            pltpu.VMEM((2, D, WP), jnp.float32),
            pltpu.SemaphoreType.DMA,
            pltpu.SemaphoreType.DMA,
            pltpu.SemaphoreType.DMA,
            pltpu.SemaphoreType.DMA,
            pltpu.SemaphoreType.DMA,
            pltpu.SemaphoreType.DMA,
        ],
        compiler_params=pltpu.CompilerParams(use_tc_tiling_on_sc=False,
                                             needs_layout_passes=False),
    )
    def body(idx_hbm, table_hbm, out_hbm, idx_v, acc, acc_t,
             si0, si1, sg0, sg1, so0, so1):
        wid = lax.axis_index("s") * 2 + lax.axis_index("c")
        iota = lax.iota(jnp.int32, L)
        zeros = jnp.zeros((L,), jnp.float32)
        sem_idx, sem_g, sem_out = (si0, si1), (sg0, sg1), (so0, so1)

        def bh(c):
            return wid * bpw + (c // H), c % H

        def fire_idx(c, j):
            b, h = bh(c)
            pltpu.async_copy(idx_hbm.at[b, :, h, :], idx_v.at[j], sem_idx[j])

        def wait_idx(c, j):
            b, h = bh(c)
            pltpu.make_async_copy(
                idx_hbm.at[b, :, h, :], idx_v.at[j], sem_idx[j]).wait()

        def zero_acc(j):
            for p in range(W):
                for half in range(D // L):
                    acc[j, p, pl.ds(half * L, L)] = zeros

        def fire_gathers(j):
            for k in range(K):
                pltpu.async_copy(table_hbm.at[idx_v.at[j, k]], acc.at[j],
                                 sem_g[j], add=True)

        def wait_gathers(j):
            for k in range(K):
                pltpu.make_async_copy(table_hbm.at[idx_v.at[j, k]],
                                      acc.at[j], sem_g[j]).wait()

        def fire_out(c, j):
            b, h = bh(c)
            pltpu.async_copy(acc_t.at[j, :, pl.ds(0, W)],
                             out_hbm.at[b, :, h, :], sem_out[j])

        def wait_out(c, j):
            b, h = bh(c)
            pltpu.make_async_copy(acc_t.at[j, :, pl.ds(0, W)],
                                  out_hbm.at[b, :, h, :], sem_out[j]).wait()

        def transpose_chunk(j):
            jvec = jnp.full((L,), j, jnp.int32)
            dvecs = [iota + half * L for half in range(D // L)]
            for p in range(W):
                pvec = jnp.full((L,), p, jnp.int32)
                for half in range(D // L):
                    val = acc[j, p, pl.ds(half * L, L)] * scale
                    plsc.store_scatter(acc_t, [jvec, dvecs[half], pvec], val)

        def half_iter(i2, j):
            c = 2 * i2 + j
            j2 = 1 - j

            def prep_next():
                wait_idx(c + 1, j2)
                zero_acc(j2)
                fire_gathers(j2)
            if j == 0:
                prep_next()
            else:
                pl.when(i2 < T // 2 - 1)(prep_next)

            wait_gathers(j)
            pl.when(c + 2 < T)(lambda: fire_idx(c + 2, j))
            pl.when(i2 >= 1)(lambda: wait_out(c - 2, j))
            transpose_chunk(j)
            fire_out(c, j)

        def step(i2, _):
            half_iter(i2, 0)
            half_iter(i2, 1)
            return 0

        fire_idx(0, 0)
        fire_idx(1, 1)
        wait_idx(0, 0)
        zero_acc(0)
        fire_gathers(0)

        lax.fori_loop(0, T // 2, step, 0, unroll=False)

        wait_out(T - 2, 0)
        wait_out(T - 1, 1)

    return body(idx, table)


def kernel(inputs, table):
    B, K, H, W = inputs.shape
    idx = inputs.astype(jnp.int32)
    return _sc_embed(idx, table, B, K, H, W)
